# baseline (speedup 1.0000x reference)
import functools

import jax
import jax.numpy as jnp
from jax import lax
from jax.experimental import pallas as pl
from jax.experimental.pallas import tpu as pltpu
from jax.experimental.pallas import tpu_sc as plsc

N = 4096
D = 512
H = 8
DH = D // H
NNZ_IN = 167772

ROW_BLK = 512

NS = 16
E_T = 10496
NNZ_PAD = NS * E_T
NCH = E_T // 128
STRIPE = 128
SLAB = STRIPE * N
SHARE = SLAB // NS
N_STRIPES = N // STRIPE


def _mask_body(rows_hbm, cols_hbm, bmv_hbm, amv_hbm, out_hbm,
               gbuf, cbuf, vbuf, abuf, idx2, val2, zbuf, zidx, zval, slab):
    cid = lax.axis_index("c")
    sid = lax.axis_index("s")
    base = sid * E_T
    pltpu.sync_copy(rows_hbm.at[pl.ds(base, E_T)], gbuf)
    pltpu.sync_copy(cols_hbm.at[pl.ds(base, E_T)], cbuf)
    pltpu.sync_copy(bmv_hbm.at[pl.ds(base, E_T)], vbuf)
    pltpu.sync_copy(amv_hbm.at[pl.ds(base, E_T)], abuf)

    def precompute(i, carry):
        sl = pl.ds(i * 16, 16)
        gbuf[sl] = gbuf[sl] * N + cbuf[sl]
        vbuf[sl] = vbuf[sl] * abuf[sl]
        return carry

    lax.fori_loop(0, E_T // 16, precompute, 0)

    def zinit(i, carry):
        zbuf[pl.ds(i * 16, 16)] = jnp.zeros((16,), jnp.float32)
        return carry

    lax.fori_loop(0, 16384 // 16, zinit, 0)
    for i in range(8):
        zidx[0, pl.ds(i * 16, 16)] = jnp.zeros((16,), jnp.int32)
        zval[0, pl.ds(i * 16, 16)] = jnp.zeros((16,), jnp.float32)

    def do_pass(p, carry):
        stripe = 2 * p + cid
        base_g = stripe * SLAB

        def zero_share(j, c2):
            pltpu.sync_copy(zbuf, slab.at[pl.ds(sid * SHARE + j * 16384, 16384)])
            return c2

        lax.fori_loop(0, SHARE // 16384, zero_share, 0)
        plsc.subcore_barrier()

        def scan(j, c2):
            for jj in range(8):
                sl = pl.ds(j * 128 + jj * 16, 16)
                lidx = gbuf[sl] - base_g
                m = (lidx >= 0) & (lidx < SLAB)
                idx2[j, pl.ds(jj * 16, 16)] = jnp.where(m, lidx, 0)
                val2[j, pl.ds(jj * 16, 16)] = jnp.where(m, vbuf[sl], 0.0)
            return c2

        lax.fori_loop(0, NCH, scan, 0)

        def scatter(j, c2):
            pltpu.sync_copy(val2.at[j], slab.at[idx2.at[j]], add=True)
            return c2

        lax.fori_loop(0, NCH, scatter, 0)
        pltpu.sync_copy(zval.at[0], slab.at[zidx.at[0]], add=True)
        plsc.subcore_barrier()
        pltpu.sync_copy(slab.at[pl.ds(sid * SHARE, SHARE)],
                        out_hbm.at[pl.ds(base_g + sid * SHARE, SHARE)])
        return carry

    lax.fori_loop(0, N_STRIPES // 2, do_pass, 0)


_mask_kernel = functools.partial(
    pl.kernel,
    out_type=jax.ShapeDtypeStruct((N * N,), jnp.float32),
    mesh=plsc.VectorSubcoreMesh(core_axis_name="c", subcore_axis_name="s"),
    scratch_types=[
        pltpu.VMEM((E_T,), jnp.int32),
        pltpu.VMEM((E_T,), jnp.int32),
        pltpu.VMEM((E_T,), jnp.float32),
        pltpu.VMEM((E_T,), jnp.float32),
        pltpu.VMEM((NCH, 128), jnp.int32),
        pltpu.VMEM((NCH, 128), jnp.float32),
        pltpu.VMEM((16384,), jnp.float32),
        pltpu.VMEM((1, 128), jnp.int32),
        pltpu.VMEM((1, 128), jnp.float32),
        pltpu.VMEM_SHARED((SLAB,), jnp.float32),
    ],
)(_mask_body)


def _build_mask(rows, cols, bmv, amv):
    pad = NNZ_PAD - NNZ_IN
    rows_p = jnp.concatenate([rows, jnp.full((pad,), -1, jnp.int32)])
    cols_p = jnp.concatenate([cols, jnp.zeros((pad,), jnp.int32)])
    bmv_p = jnp.concatenate([bmv, jnp.zeros((pad,), jnp.float32)])
    amv_p = jnp.concatenate([amv, jnp.zeros((pad,), jnp.float32)])
    return _mask_kernel(rows_p, cols_p, bmv_p, amv_p).reshape(N, N)


def _ln(h, g, b):
    mu = jnp.mean(h, axis=-1, keepdims=True)
    var = jnp.mean((h - mu) ** 2, axis=-1, keepdims=True)
    return (h - mu) / jnp.sqrt(var + 1e-5) * g + b


def _qkv_body(x_ref, g_ref, b_ref, wq_ref, bq_ref, wk_ref, bk_ref, wv_ref, bv_ref,
              q_ref, k_ref, v_ref):
    xn = _ln(x_ref[...], g_ref[...], b_ref[...])
    q_ref[...] = jnp.dot(xn, wq_ref[...], preferred_element_type=jnp.float32) + bq_ref[...]
    k_ref[...] = jnp.dot(xn, wk_ref[...], preferred_element_type=jnp.float32) + bk_ref[...]
    v_ref[...] = jnp.dot(xn, wv_ref[...], preferred_element_type=jnp.float32) + bv_ref[...]


def _qkv(x, ln1_g, ln1_b, Wq, bq, Wk, bk, Wv, bv):
    grid = (N // ROW_BLK,)
    row_spec = pl.BlockSpec((ROW_BLK, D), lambda i: (i, 0))
    full_w = pl.BlockSpec((D, D), lambda i: (0, 0))
    full_b = pl.BlockSpec((D,), lambda i: (0,))
    return pl.pallas_call(
        _qkv_body,
        grid=grid,
        in_specs=[row_spec, full_b, full_b, full_w, full_b, full_w, full_b, full_w, full_b],
        out_specs=[row_spec, row_spec, row_spec],
        out_shape=[jax.ShapeDtypeStruct((N, D), jnp.float32)] * 3,
    )(x, ln1_g, ln1_b, Wq, bq, Wk, bk, Wv, bv)


def _tail_body(x_ref, ao_ref, wo_ref, bo_ref, g_ref, b_ref, w1_ref, b1_ref, w2_ref, b2_ref,
               out_ref):
    h = x_ref[...] + jnp.dot(ao_ref[...], wo_ref[...], preferred_element_type=jnp.float32) + bo_ref[...]
    hn = _ln(h, g_ref[...], b_ref[...])
    up = jax.nn.gelu(jnp.dot(hn, w1_ref[...], preferred_element_type=jnp.float32) + b1_ref[...])
    out_ref[...] = h + jnp.dot(up, w2_ref[...], preferred_element_type=jnp.float32) + b2_ref[...]


def _tail(x, attnout, Wo, bo, ln2_g, ln2_b, W1, b1, W2, b2):
    grid = (N // ROW_BLK,)
    row_spec = pl.BlockSpec((ROW_BLK, D), lambda i: (i, 0))
    return pl.pallas_call(
        _tail_body,
        grid=grid,
        in_specs=[row_spec, row_spec,
                  pl.BlockSpec((D, D), lambda i: (0, 0)),
                  pl.BlockSpec((D,), lambda i: (0,)),
                  pl.BlockSpec((D,), lambda i: (0,)),
                  pl.BlockSpec((D,), lambda i: (0,)),
                  pl.BlockSpec((D, 2 * D), lambda i: (0, 0)),
                  pl.BlockSpec((2 * D,), lambda i: (0,)),
                  pl.BlockSpec((2 * D, D), lambda i: (0, 0)),
                  pl.BlockSpec((D,), lambda i: (0,))],
        out_specs=row_spec,
        out_shape=jax.ShapeDtypeStruct((N, D), jnp.float32),
    )(x, attnout, Wo, bo, ln2_g, ln2_b, W1, b1, W2, b2)


BM = 256


def _attn_body(q_ref, k_ref, v_ref, m_ref, o_ref):
    q = q_ref[0]
    k = k_ref[0]
    v = v_ref[0]
    s = jax.lax.dot_general(q, k, (((1,), (1,)), ((), ())),
                            preferred_element_type=jnp.float32) * (1.0 / 8.0)
    mx = jnp.max(s, axis=1, keepdims=True)
    w = jnp.exp(s - mx) * m_ref[...]
    denom = jnp.sum(w, axis=1, keepdims=True) + 1e-9
    p = w / denom
    o_ref[0] = jax.lax.dot_general(p, v, (((1,), (0,)), ((), ())),
                                   preferred_element_type=jnp.float32)


def _attn(qh, kh, vh, mask):
    grid = (N // BM, H)
    return pl.pallas_call(
        _attn_body,
        grid=grid,
        in_specs=[pl.BlockSpec((1, BM, DH), lambda i, h: (h, i, 0)),
                  pl.BlockSpec((1, N, DH), lambda i, h: (h, 0, 0)),
                  pl.BlockSpec((1, N, DH), lambda i, h: (h, 0, 0)),
                  pl.BlockSpec((BM, N), lambda i, h: (i, 0))],
        out_specs=pl.BlockSpec((1, BM, DH), lambda i, h: (h, i, 0)),
        out_shape=jax.ShapeDtypeStruct((H, N, DH), jnp.float32),
    )(qh, kh, vh, mask)


def kernel(x, batch_mask_indices, batch_mask_values, attention_mask_indices, attention_mask_values, ln1_g, ln1_b, Wq, bq, Wk, bk, Wv, bv, Wo, bo, ln2_g, ln2_b, W1, b1, W2, b2):
    rows = attention_mask_indices[0]
    cols = attention_mask_indices[1]
    mask = _build_mask(rows, cols, batch_mask_values, attention_mask_values)
    q, k, v = _qkv(x, ln1_g, ln1_b, Wq, bq, Wk, bk, Wv, bv)
    qh = q.reshape(N, H, DH).transpose(1, 0, 2)
    kh = k.reshape(N, H, DH).transpose(1, 0, 2)
    vh = v.reshape(N, H, DH).transpose(1, 0, 2)
    out = _attn(qh, kh, vh, mask).transpose(1, 0, 2)
    return _tail(x, out.reshape(N, D), Wo, bo, ln2_g, ln2_b, W1, b1, W2, b2)


# speedup vs baseline: 3.5224x; 3.5224x over previous
import functools

import jax
import jax.numpy as jnp
from jax import lax
from jax.experimental import pallas as pl
from jax.experimental.pallas import tpu as pltpu
from jax.experimental.pallas import tpu_sc as plsc

N = 4096
D = 512
H = 8
DH = D // H
NNZ_IN = 167772

ROW_BLK = 512

NS = 16
E_T = 10496
NNZ_PAD = NS * E_T
NCH = E_T // 128
STRIPE = 128
SLAB = STRIPE * N
SHARE = SLAB // NS
N_STRIPES = N // STRIPE


def _mask_body(rows_hbm, cols_hbm, bmv_hbm, amv_hbm, out_hbm,
               gbuf, cbuf, vbuf, abuf, idxc, valc, zbuf, zidx, zval, slab):
    cid = lax.axis_index("c")
    sid = lax.axis_index("s")
    base = sid * E_T
    pltpu.sync_copy(rows_hbm.at[pl.ds(base, E_T)], gbuf)
    pltpu.sync_copy(cols_hbm.at[pl.ds(base, E_T)], cbuf)
    pltpu.sync_copy(bmv_hbm.at[pl.ds(base, E_T)], vbuf)
    pltpu.sync_copy(amv_hbm.at[pl.ds(base, E_T)], abuf)

    def precompute(i, carry):
        sl = pl.ds(i * 16, 16)
        gbuf[sl] = gbuf[sl] * N + cbuf[sl]
        vbuf[sl] = vbuf[sl] * abuf[sl]
        return carry

    lax.fori_loop(0, E_T // 16, precompute, 0)

    def zinit(i, carry):
        zbuf[pl.ds(i * 16, 16)] = jnp.zeros((16,), jnp.float32)
        return carry

    lax.fori_loop(0, 16384 // 16, zinit, 0)
    for i in range(8):
        zidx[0, pl.ds(i * 16, 16)] = jnp.zeros((16,), jnp.int32)
        zval[0, pl.ds(i * 16, 16)] = jnp.zeros((16,), jnp.float32)

    def do_pass(p, carry):
        stripe = 2 * p + cid
        base_g = stripe * SLAB

        def zero_share(j, c2):
            pltpu.sync_copy(zbuf, slab.at[pl.ds(sid * SHARE + j * 16384, 16384)])
            return c2

        lax.fori_loop(0, SHARE // 16384, zero_share, 0)
        plsc.subcore_barrier()

        dump = sid * SHARE + lax.iota(jnp.int32, 16)

        def scan(j, c2):
            for jj in range(8):
                sl = pl.ds(j * 128 + jj * 16, 16)
                lidx = gbuf[sl] - base_g
                m = (lidx >= 0) & (lidx < SLAB)
                idxc[j, pl.ds(jj * 16, 16)] = jnp.where(m, lidx, dump)
                valc[j, pl.ds(jj * 16, 16)] = jnp.where(m, vbuf[sl], 0.0)
            return c2

        lax.fori_loop(0, NCH, scan, 0)

        def scatter(j, c2):
            pltpu.sync_copy(valc.at[j], slab.at[idxc.at[j]], add=True)
            return c2

        lax.fori_loop(0, NCH, scatter, 0)
        pltpu.sync_copy(zval.at[0], slab.at[zidx.at[0]], add=True)
        plsc.subcore_barrier()
        pltpu.sync_copy(slab.at[pl.ds(sid * SHARE, SHARE)],
                        out_hbm.at[pl.ds(base_g + sid * SHARE, SHARE)])
        return carry

    lax.fori_loop(0, N_STRIPES // 2, do_pass, 0)


_mask_kernel = functools.partial(
    pl.kernel,
    out_type=jax.ShapeDtypeStruct((N * N,), jnp.float32),
    mesh=plsc.VectorSubcoreMesh(core_axis_name="c", subcore_axis_name="s"),
    scratch_types=[
        pltpu.VMEM((E_T,), jnp.int32),
        pltpu.VMEM((E_T,), jnp.int32),
        pltpu.VMEM((E_T,), jnp.float32),
        pltpu.VMEM((E_T,), jnp.float32),
        pltpu.VMEM((NCH, 128), jnp.int32),
        pltpu.VMEM((NCH, 128), jnp.float32),
        pltpu.VMEM((16384,), jnp.float32),
        pltpu.VMEM((1, 128), jnp.int32),
        pltpu.VMEM((1, 128), jnp.float32),
        pltpu.VMEM_SHARED((SLAB,), jnp.float32),
    ],
)(_mask_body)


def _build_mask(rows, cols, bmv, amv):
    pad = NNZ_PAD - NNZ_IN
    rows_p = jnp.concatenate([rows, jnp.full((pad,), -1, jnp.int32)])
    cols_p = jnp.concatenate([cols, jnp.zeros((pad,), jnp.int32)])
    bmv_p = jnp.concatenate([bmv, jnp.zeros((pad,), jnp.float32)])
    amv_p = jnp.concatenate([amv, jnp.zeros((pad,), jnp.float32)])
    return _mask_kernel(rows_p, cols_p, bmv_p, amv_p).reshape(N, N)


def _ln(h, g, b):
    mu = jnp.mean(h, axis=-1, keepdims=True)
    var = jnp.mean((h - mu) ** 2, axis=-1, keepdims=True)
    return (h - mu) / jnp.sqrt(var + 1e-5) * g + b


def _qkv_body(x_ref, g_ref, b_ref, wq_ref, bq_ref, wk_ref, bk_ref, wv_ref, bv_ref,
              q_ref, k_ref, v_ref):
    xn = _ln(x_ref[...], g_ref[...], b_ref[...])
    q_ref[...] = jnp.dot(xn, wq_ref[...], preferred_element_type=jnp.float32) + bq_ref[...]
    k_ref[...] = jnp.dot(xn, wk_ref[...], preferred_element_type=jnp.float32) + bk_ref[...]
    v_ref[...] = jnp.dot(xn, wv_ref[...], preferred_element_type=jnp.float32) + bv_ref[...]


def _qkv(x, ln1_g, ln1_b, Wq, bq, Wk, bk, Wv, bv):
    grid = (N // ROW_BLK,)
    row_spec = pl.BlockSpec((ROW_BLK, D), lambda i: (i, 0))
    full_w = pl.BlockSpec((D, D), lambda i: (0, 0))
    full_b = pl.BlockSpec((D,), lambda i: (0,))
    return pl.pallas_call(
        _qkv_body,
        grid=grid,
        in_specs=[row_spec, full_b, full_b, full_w, full_b, full_w, full_b, full_w, full_b],
        out_specs=[row_spec, row_spec, row_spec],
        out_shape=[jax.ShapeDtypeStruct((N, D), jnp.float32)] * 3,
    )(x, ln1_g, ln1_b, Wq, bq, Wk, bk, Wv, bv)


def _tail_body(x_ref, ao_ref, wo_ref, bo_ref, g_ref, b_ref, w1_ref, b1_ref, w2_ref, b2_ref,
               out_ref):
    h = x_ref[...] + jnp.dot(ao_ref[...], wo_ref[...], preferred_element_type=jnp.float32) + bo_ref[...]
    hn = _ln(h, g_ref[...], b_ref[...])
    up = jax.nn.gelu(jnp.dot(hn, w1_ref[...], preferred_element_type=jnp.float32) + b1_ref[...])
    out_ref[...] = h + jnp.dot(up, w2_ref[...], preferred_element_type=jnp.float32) + b2_ref[...]


def _tail(x, attnout, Wo, bo, ln2_g, ln2_b, W1, b1, W2, b2):
    grid = (N // ROW_BLK,)
    row_spec = pl.BlockSpec((ROW_BLK, D), lambda i: (i, 0))
    return pl.pallas_call(
        _tail_body,
        grid=grid,
        in_specs=[row_spec, row_spec,
                  pl.BlockSpec((D, D), lambda i: (0, 0)),
                  pl.BlockSpec((D,), lambda i: (0,)),
                  pl.BlockSpec((D,), lambda i: (0,)),
                  pl.BlockSpec((D,), lambda i: (0,)),
                  pl.BlockSpec((D, 2 * D), lambda i: (0, 0)),
                  pl.BlockSpec((2 * D,), lambda i: (0,)),
                  pl.BlockSpec((2 * D, D), lambda i: (0, 0)),
                  pl.BlockSpec((D,), lambda i: (0,))],
        out_specs=row_spec,
        out_shape=jax.ShapeDtypeStruct((N, D), jnp.float32),
    )(x, attnout, Wo, bo, ln2_g, ln2_b, W1, b1, W2, b2)


BM = 512


def _attn_body(q_ref, k_ref, v_ref, m_ref, o_ref):
    q = q_ref[0]
    k = k_ref[0]
    v = v_ref[0]
    s = jax.lax.dot_general(q, k, (((1,), (1,)), ((), ())),
                            preferred_element_type=jnp.float32) * (1.0 / 8.0)
    mx = jnp.max(s, axis=1, keepdims=True)
    w = jnp.exp(s - mx) * m_ref[...]
    denom = jnp.sum(w, axis=1, keepdims=True) + 1e-9
    p = w / denom
    o_ref[0] = jax.lax.dot_general(p, v, (((1,), (0,)), ((), ())),
                                   preferred_element_type=jnp.float32)


def _attn(qh, kh, vh, mask):
    grid = (N // BM, H)
    return pl.pallas_call(
        _attn_body,
        grid=grid,
        in_specs=[pl.BlockSpec((1, BM, DH), lambda i, h: (h, i, 0)),
                  pl.BlockSpec((1, N, DH), lambda i, h: (h, 0, 0)),
                  pl.BlockSpec((1, N, DH), lambda i, h: (h, 0, 0)),
                  pl.BlockSpec((BM, N), lambda i, h: (i, 0))],
        out_specs=pl.BlockSpec((1, BM, DH), lambda i, h: (h, i, 0)),
        out_shape=jax.ShapeDtypeStruct((H, N, DH), jnp.float32),
    )(qh, kh, vh, mask)


def kernel(x, batch_mask_indices, batch_mask_values, attention_mask_indices, attention_mask_values, ln1_g, ln1_b, Wq, bq, Wk, bk, Wv, bv, Wo, bo, ln2_g, ln2_b, W1, b1, W2, b2):
    rows = attention_mask_indices[0]
    cols = attention_mask_indices[1]
    mask = _build_mask(rows, cols, batch_mask_values, attention_mask_values)
    q, k, v = _qkv(x, ln1_g, ln1_b, Wq, bq, Wk, bk, Wv, bv)
    qh = q.reshape(N, H, DH).transpose(1, 0, 2)
    kh = k.reshape(N, H, DH).transpose(1, 0, 2)
    vh = v.reshape(N, H, DH).transpose(1, 0, 2)
    out = _attn(qh, kh, vh, mask).transpose(1, 0, 2)
    return _tail(x, out.reshape(N, D), Wo, bo, ln2_g, ln2_b, W1, b1, W2, b2)
